# baseline (speedup 1.0000x reference)
import jax
import jax.numpy as jnp
from jax import lax
from jax.experimental import pallas as pl
from jax.experimental.pallas import tpu as pltpu

_Q = 1000
_K = 4
_BB = 8
_NBUF = 4


def _body(qm1_ref, r3_ref, qpos_ref, kf3_ref, o_hbm, buf, sems):
    B = qm1_ref.shape[0]
    qpos = qpos_ref[...]
    kf3 = kf3_ref[...]
    nouter = B // (_BB * _NBUF)

    def compute(g, b):
        qm1 = qm1_ref[pl.ds(g * _BB, _BB), :][:, :, None]
        r3 = r3_ref[pl.ds(g * _BB, _BB), :][:, :, None]
        w = 1.0 - jnp.abs(kf3 - r3)
        buf[b] = jnp.where(qpos == qm1, w, 0.0)

    def step(g2, carry):
        for b in range(_NBUF):
            g = g2 * _NBUF + b

            @pl.when(g2 >= 1)
            def _():
                pltpu.make_async_copy(
                    buf.at[b], o_hbm.at[pl.ds((g - _NBUF) * _BB, _BB)],
                    sems.at[b]).wait()

            compute(g, b)
            pltpu.make_async_copy(
                buf.at[b], o_hbm.at[pl.ds(g * _BB, _BB)], sems.at[b]).start()
        return carry

    lax.fori_loop(0, nouter, step, 0)

    for b in range(_NBUF):
        g = (nouter - 1) * _NBUF + b
        pltpu.make_async_copy(
            buf.at[b], o_hbm.at[pl.ds(g * _BB, _BB)], sems.at[b]).wait()


def kernel(question_ids, responses):
    B, S = responses.shape
    qm1 = question_ids.astype(jnp.int32) - 1
    r3 = responses.astype(jnp.float32) * (1.0 / (_K - 1))
    col = jnp.arange(_K * _Q, dtype=jnp.int32)
    qpos = (col % _Q).reshape(1, 1, _K * _Q)
    kf3 = ((col // _Q).astype(jnp.float32) * (1.0 / (_K - 1))).reshape(1, 1, _K * _Q)
    return pl.pallas_call(
        _body,
        out_shape=jax.ShapeDtypeStruct((B, S, _K * _Q), jnp.float32),
        in_specs=[
            pl.BlockSpec(memory_space=pltpu.VMEM),
            pl.BlockSpec(memory_space=pltpu.VMEM),
            pl.BlockSpec(memory_space=pltpu.VMEM),
            pl.BlockSpec(memory_space=pltpu.VMEM),
        ],
        out_specs=pl.BlockSpec(memory_space=pl.ANY),
        scratch_shapes=[
            pltpu.VMEM((_NBUF, _BB, S, _K * _Q), jnp.float32),
            pltpu.SemaphoreType.DMA((_NBUF,)),
        ],
    )(qm1, r3, qpos, kf3)
